# Initial kernel scaffold; baseline (speedup 1.0000x reference)
import jax
import jax.numpy as jnp
from jax.experimental import pallas as pl


def kernel(xyz, feat):
    raise NotImplementedError("write your pallas kernel here")


# speedup vs baseline: 5.8113x; 5.8113x over previous
import functools

import jax
import jax.numpy as jnp
from jax import lax
from jax.experimental import pallas as pl
from jax.experimental.pallas import tpu as pltpu

B = 16
N = 8192
G = 512
K = 16
C_SLOTS = 5
RB = 16
NCHUNK = N // 128


def _fps_body(x_ref, y_ref, z_ref, idx_ref, lx_ref, ly_ref, lz_ref, dist_ref):
    X = x_ref[...]
    Y = y_ref[...]
    Z = z_ref[...]
    lane = lax.broadcasted_iota(jnp.int32, (B, N), 1)
    gcol = lax.broadcasted_iota(jnp.int32, (B, G), 1)

    lx0 = X[:, 0:1]
    ly0 = Y[:, 0:1]
    lz0 = Z[:, 0:1]
    idx_ref[...] = jnp.zeros((B, G), jnp.int32)
    lx_ref[...] = jnp.where(gcol == 0, lx0, 0.0)
    ly_ref[...] = jnp.where(gcol == 0, ly0, 0.0)
    lz_ref[...] = jnp.where(gcol == 0, lz0, 0.0)
    dist_ref[...] = jnp.full((B, N), jnp.inf, jnp.float32)

    def body(i, carry):
        lx, ly, lz = carry
        dx = X - lx
        dy = Y - ly
        dz = Z - lz
        d = (dx * dx + dy * dy) + dz * dz
        dm = jnp.minimum(dist_ref[...], d)
        dist_ref[...] = dm
        m = jnp.max(dm, axis=1, keepdims=True)
        nxt = jnp.max(jnp.where(dm == m, lane, -1), axis=1, keepdims=True)
        sel = lane == nxt
        nlx = jnp.sum(jnp.where(sel, X, 0.0), axis=1, keepdims=True)
        nly = jnp.sum(jnp.where(sel, Y, 0.0), axis=1, keepdims=True)
        nlz = jnp.sum(jnp.where(sel, Z, 0.0), axis=1, keepdims=True)
        hit = gcol == i
        idx_ref[...] = jnp.where(hit, nxt, idx_ref[...])
        lx_ref[...] = jnp.where(hit, nlx, lx_ref[...])
        ly_ref[...] = jnp.where(hit, nly, ly_ref[...])
        lz_ref[...] = jnp.where(hit, nlz, lz_ref[...])
        return (nlx, nly, nlz)

    lax.fori_loop(1, G, body, (lx0, ly0, lz0))


def _run_fps(x, y, z):
    out_shapes = (
        jax.ShapeDtypeStruct((B, G), jnp.int32),
        jax.ShapeDtypeStruct((B, G), jnp.float32),
        jax.ShapeDtypeStruct((B, G), jnp.float32),
        jax.ShapeDtypeStruct((B, G), jnp.float32),
    )
    return pl.pallas_call(
        _fps_body,
        out_shape=out_shapes,
        scratch_shapes=[pltpu.VMEM((B, N), jnp.float32)],
    )(x, y, z)


def _topk_body(x_ref, y_ref, z_ref, lcx_ref, lcy_ref, lcz_ref, knn_ref):
    INF = jnp.float32(jnp.inf)
    BIGI = jnp.int32(1 << 30)
    lcx = lcx_ref[0, 0]
    lcy = lcy_ref[0, 0]
    lcz = lcz_ref[0, 0]
    lane = lax.broadcasted_iota(jnp.int32, (RB, 128), 1)

    sv0 = [jnp.full((RB, 128), INF, jnp.float32) for _ in range(C_SLOTS)]
    si0 = [jnp.full((RB, 128), BIGI, jnp.int32) for _ in range(C_SLOTS)]

    def chunk(j, carry):
        sv = list(carry[0])
        si = list(carry[1])
        xb = x_ref[0, pl.ds(j, 1), :]
        yb = y_ref[0, pl.ds(j, 1), :]
        zb = z_ref[0, pl.ds(j, 1), :]
        dx = lcx - xb
        dy = lcy - yb
        dz = lcz - zb
        cv = (dx * dx + dy * dy) + dz * dz
        ci = lane + j * 128
        for k in range(C_SLOTS):
            swap = cv < sv[k]
            nsv = jnp.where(swap, cv, sv[k])
            ncv = jnp.where(swap, sv[k], cv)
            nsi = jnp.where(swap, ci, si[k])
            nci = jnp.where(swap, si[k], ci)
            sv[k], cv = nsv, ncv
            si[k], ci = nsi, nci
        return (tuple(sv), tuple(si))

    sv, si = lax.fori_loop(0, NCHUNK, chunk, (tuple(sv0), tuple(si0)))

    kcol = lax.broadcasted_iota(jnp.int32, (RB, K), 1)

    def extract(k, carry):
        sv = list(carry[0])
        si = list(carry[1])
        acc = carry[2]
        mv = sv[0]
        for t in range(1, C_SLOTS):
            mv = jnp.minimum(mv, sv[t])
        m = jnp.min(mv, axis=1, keepdims=True)
        icand = jnp.full((RB, 128), BIGI, jnp.int32)
        for t in range(C_SLOTS):
            icand = jnp.minimum(icand, jnp.where(sv[t] == m, si[t], BIGI))
        nxt = jnp.min(icand, axis=1, keepdims=True)
        acc = jnp.where(kcol == k, nxt, acc)
        for t in range(C_SLOTS):
            sv[t] = jnp.where(si[t] == nxt, INF, sv[t])
        return (tuple(sv), tuple(si), acc)

    acc0 = jnp.zeros((RB, K), jnp.int32)
    _, _, acc = lax.fori_loop(0, K, extract, (sv, si, acc0))
    knn_ref[0, 0] = acc


def _run_topk(x3, y3, z3, lcx4, lcy4, lcz4):
    grid = (B, G // RB)
    xspec = pl.BlockSpec((1, NCHUNK, 128), lambda b, r: (b, 0, 0))
    lspec = pl.BlockSpec((1, 1, RB, 1), lambda b, r: (b, r, 0, 0))
    return pl.pallas_call(
        _topk_body,
        grid=grid,
        in_specs=[xspec, xspec, xspec, lspec, lspec, lspec],
        out_specs=pl.BlockSpec((1, 1, RB, K), lambda b, r: (b, r, 0, 0)),
        out_shape=jax.ShapeDtypeStruct((B, G // RB, RB, K), jnp.int32),
    )(x3, y3, z3, lcx4, lcy4, lcz4)


@jax.jit
def kernel(xyz, feat):
    x = xyz[:, :, 0]
    y = xyz[:, :, 1]
    z = xyz[:, :, 2]
    fps_idx, lcx, lcy, lcz = _run_fps(x, y, z)
    lc_xyz = jnp.stack([lcx, lcy, lcz], axis=-1)

    x3 = x.reshape(B, NCHUNK, 128)
    y3 = y.reshape(B, NCHUNK, 128)
    z3 = z.reshape(B, NCHUNK, 128)
    lcx4 = lcx.reshape(B, G // RB, RB, 1)
    lcy4 = lcy.reshape(B, G // RB, RB, 1)
    lcz4 = lcz.reshape(B, G // RB, RB, 1)
    knn_idx = _run_topk(x3, y3, z3, lcx4, lcy4, lcz4).reshape(B, G, K)

    lc_feat = jnp.take_along_axis(
        feat, fps_idx[:, :, None], axis=1)
    knn_xyz = jax.vmap(lambda p, i: p[i])(xyz, knn_idx)
    knn_feat = jax.vmap(lambda f, i: f[i])(feat, knn_idx)
    return (lc_xyz, lc_feat, knn_xyz, knn_feat)


# speedup vs baseline: 18.9314x; 3.2577x over previous
import functools

import jax
import jax.numpy as jnp
from jax import lax
from jax.experimental import pallas as pl
from jax.experimental.pallas import tpu as pltpu

B = 16
N = 8192
G = 512
K = 16
C_SLOTS = 4
RB = 16
NCHUNK = N // 128


def _fps_body(x_ref, y_ref, z_ref, idx_ref, lx_ref, ly_ref, lz_ref, dist_ref):
    X = x_ref[...]
    Y = y_ref[...]
    Z = z_ref[...]
    lane = lax.broadcasted_iota(jnp.int32, (B, N), 1)
    gcol = lax.broadcasted_iota(jnp.int32, (B, G), 1)

    lx0 = X[:, 0:1]
    ly0 = Y[:, 0:1]
    lz0 = Z[:, 0:1]
    idx_ref[...] = jnp.zeros((B, G), jnp.int32)
    lx_ref[...] = jnp.where(gcol == 0, lx0, 0.0)
    ly_ref[...] = jnp.where(gcol == 0, ly0, 0.0)
    lz_ref[...] = jnp.where(gcol == 0, lz0, 0.0)
    dist_ref[...] = jnp.full((B, N), jnp.inf, jnp.float32)

    def body(i, carry):
        lx, ly, lz = carry
        dx = X - lx
        dy = Y - ly
        dz = Z - lz
        d = (dx * dx + dy * dy) + dz * dz
        dm = jnp.minimum(dist_ref[...], d)
        dist_ref[...] = dm
        m = jnp.max(dm, axis=1, keepdims=True)
        nxt = jnp.max(jnp.where(dm == m, lane, -1), axis=1, keepdims=True)
        sel = lane == nxt
        nlx = jnp.sum(jnp.where(sel, X, 0.0), axis=1, keepdims=True)
        nly = jnp.sum(jnp.where(sel, Y, 0.0), axis=1, keepdims=True)
        nlz = jnp.sum(jnp.where(sel, Z, 0.0), axis=1, keepdims=True)
        hit = gcol == i
        idx_ref[...] = jnp.where(hit, nxt, idx_ref[...])
        lx_ref[...] = jnp.where(hit, nlx, lx_ref[...])
        ly_ref[...] = jnp.where(hit, nly, ly_ref[...])
        lz_ref[...] = jnp.where(hit, nlz, lz_ref[...])
        return (nlx, nly, nlz)

    lax.fori_loop(1, G, body, (lx0, ly0, lz0))


def _run_fps(x, y, z):
    out_shapes = (
        jax.ShapeDtypeStruct((B, G), jnp.int32),
        jax.ShapeDtypeStruct((B, G), jnp.float32),
        jax.ShapeDtypeStruct((B, G), jnp.float32),
        jax.ShapeDtypeStruct((B, G), jnp.float32),
    )
    return pl.pallas_call(
        _fps_body,
        out_shape=out_shapes,
        scratch_shapes=[pltpu.VMEM((B, N), jnp.float32)],
    )(x, y, z)


def _ce(a, b):
    av, ai = a
    bv, bi = b
    sw = (bv < av) | ((bv == av) & (bi < ai))
    lo = (jnp.where(sw, bv, av), jnp.where(sw, bi, ai))
    hi = (jnp.where(sw, av, bv), jnp.where(sw, ai, bi))
    return lo, hi


def _bitonic_sort(seq):
    n = len(seq)
    if n == 1:
        return seq
    half = n // 2
    for i in range(half):
        seq[i], seq[i + half] = _ce(seq[i], seq[i + half])
    return _bitonic_sort(seq[:half]) + _bitonic_sort(seq[half:])


def _merge_sorted(a, b, m):
    k = len(a)
    lo, hi = [], []
    for i in range(k):
        l, h = _ce(a[i], b[k - 1 - i])
        lo.append(l)
        hi.append(h)
    out = _bitonic_sort(lo)
    if m > k:
        out = out + _bitonic_sort(hi)
    return out[:m]


def _topk_body(x_ref, y_ref, z_ref, lcx_ref, lcy_ref, lcz_ref, knn_ref):
    INF = jnp.float32(jnp.inf)
    BIGI = jnp.int32(1 << 30)
    lcx = lcx_ref[0, 0]
    lcy = lcy_ref[0, 0]
    lcz = lcz_ref[0, 0]
    lane = lax.broadcasted_iota(jnp.int32, (RB, 128), 1)

    sv = [jnp.full((RB, 128), INF, jnp.float32) for _ in range(C_SLOTS)]
    si = [jnp.full((RB, 128), BIGI, jnp.int32) for _ in range(C_SLOTS)]

    for j in range(NCHUNK):
        xb = x_ref[0, j:j + 1, :]
        yb = y_ref[0, j:j + 1, :]
        zb = z_ref[0, j:j + 1, :]
        dx = lcx - xb
        dy = lcy - yb
        dz = lcz - zb
        cv = (dx * dx + dy * dy) + dz * dz
        ci = lane + j * 128
        for k in range(C_SLOTS):
            swap = cv < sv[k]
            nsv = jnp.where(swap, cv, sv[k])
            ncv = jnp.where(swap, sv[k], cv)
            nsi = jnp.where(swap, ci, si[k])
            nci = jnp.where(swap, si[k], ci)
            sv[k], cv = nsv, ncv
            si[k], ci = nsi, nci

    cur = list(zip(sv, si))
    width = 128
    while width > 1:
        half = width // 2
        rot = [(jnp.roll(v, -half, axis=1), jnp.roll(i, -half, axis=1))
               for (v, i) in cur]
        m = min(2 * len(cur), K)
        cur = _merge_sorted(cur, rot, m)
        width = half

    kcol = lax.broadcasted_iota(jnp.int32, (RB, K), 1)
    acc = jnp.zeros((RB, K), jnp.int32)
    for k in range(K):
        acc = jnp.where(kcol == k, cur[k][1][:, 0:1], acc)
    knn_ref[0, 0] = acc


def _run_topk(x3, y3, z3, lcx4, lcy4, lcz4):
    grid = (B, G // RB)
    xspec = pl.BlockSpec((1, NCHUNK, 128), lambda b, r: (b, 0, 0))
    lspec = pl.BlockSpec((1, 1, RB, 1), lambda b, r: (b, r, 0, 0))
    return pl.pallas_call(
        _topk_body,
        grid=grid,
        in_specs=[xspec, xspec, xspec, lspec, lspec, lspec],
        out_specs=pl.BlockSpec((1, 1, RB, K), lambda b, r: (b, r, 0, 0)),
        out_shape=jax.ShapeDtypeStruct((B, G // RB, RB, K), jnp.int32),
    )(x3, y3, z3, lcx4, lcy4, lcz4)


@jax.jit
def kernel(xyz, feat):
    x = xyz[:, :, 0]
    y = xyz[:, :, 1]
    z = xyz[:, :, 2]
    fps_idx, lcx, lcy, lcz = _run_fps(x, y, z)
    lc_xyz = jnp.stack([lcx, lcy, lcz], axis=-1)

    x3 = x.reshape(B, NCHUNK, 128)
    y3 = y.reshape(B, NCHUNK, 128)
    z3 = z.reshape(B, NCHUNK, 128)
    lcx4 = lcx.reshape(B, G // RB, RB, 1)
    lcy4 = lcy.reshape(B, G // RB, RB, 1)
    lcz4 = lcz.reshape(B, G // RB, RB, 1)
    knn_idx = _run_topk(x3, y3, z3, lcx4, lcy4, lcz4).reshape(B, G, K)

    lc_feat = jnp.take_along_axis(
        feat, fps_idx[:, :, None], axis=1)
    knn_xyz = jax.vmap(lambda p, i: p[i])(xyz, knn_idx)
    knn_feat = jax.vmap(lambda f, i: f[i])(feat, knn_idx)
    return (lc_xyz, lc_feat, knn_xyz, knn_feat)
